# baseline (speedup 1.0000x reference)
import functools

import jax
import jax.numpy as jnp
from jax import lax
from jax.experimental import pallas as pl
from jax.experimental.pallas import tpu as pltpu
from jax.experimental.pallas import tpu_sc as plsc

_NC = 2
_NS = 16
_NW = _NC * _NS
_L = 16
_CH = 128

_F1 = (1.0, -1.0 / 6, 1.0 / 120, -1.0 / 5040, 1.0 / 362880, -1.0 / 39916800)
_F2 = (0.5, -1.0 / 24, 1.0 / 720, -1.0 / 40320, 1.0 / 3628800,
       -1.0 / 479001600)


def _poly(coeffs, x):
    acc = jnp.full((_L,), coeffs[-1], jnp.float32)
    for c in coeffs[-2::-1]:
        acc = acc * x + jnp.float32(c)
    return acc


@functools.partial(jax.jit, static_argnums=(2,))
def _gather_expmap(indices, table, bpw):
    b = indices.shape[0]
    cpw = bpw // _CH
    mesh = plsc.VectorSubcoreMesh(
        core_axis_name="c", subcore_axis_name="s",
        num_cores=_NC, num_subcores=_NS)

    @functools.partial(
        pl.kernel,
        mesh=mesh,
        out_type=jax.ShapeDtypeStruct((b, 3, 4), jnp.float32),
        scratch_types=[
            pltpu.VMEM((bpw,), jnp.int32),
            pltpu.VMEM((cpw, _CH, 8), jnp.float32),
            pltpu.VMEM((bpw, 3, 4), jnp.float32),
            pltpu.SemaphoreType.DMA,
        ],
        compiler_params=pltpu.CompilerParams(
            needs_layout_passes=False, use_tc_tiling_on_sc=False),
    )
    def k(idx_hbm, table_hbm, out_hbm, idx_v, rows_v, out_v, sem):
        wid = lax.axis_index("s") * _NC + lax.axis_index("c")
        base = wid * bpw
        pltpu.sync_copy(idx_hbm.at[pl.ds(base, bpw)], idx_v)
        copies = [
            pltpu.async_copy(
                table_hbm.at[idx_v.at[pl.ds(j * _CH, _CH)]],
                rows_v.at[j], sem)
            for j in range(cpw)
        ]
        for j in range(cpw):
            copies[j].wait()
            jidx = jnp.full((_L,), j, jnp.int32)
            for s in range(_CH // _L):
                ridx = jnp.arange(_L, dtype=jnp.int32) + (s * _L)
                orow = ridx + (j * _CH)

                def col(kk):
                    cidx = jnp.full((_L,), kk, jnp.int32)
                    return plsc.load_gather(rows_v, [jidx, ridx, cidx])

                tx, ty, tz = col(0), col(1), col(2)
                wx, wy, wz = col(3), col(4), col(5)
                nr = wx * wx + wy * wy + wz * wz
                x = jnp.maximum(nr, jnp.float32(1e-4))
                f1 = _poly(_F1, x)
                f2 = _poly(_F2, x)
                diag = 1.0 - f2 * nr
                cwx = f2 * wx
                cxy = cwx * wy
                cxz = cwx * wz
                cyz = f2 * wy * wz
                sx = f1 * wx
                sy = f1 * wy
                sz = f1 * wz
                cols = (
                    diag + cwx * wx, cxy - sz, cxz + sy, tx,
                    cxy + sz, diag + f2 * wy * wy, cyz - sx, ty,
                    cxz - sy, cyz + sx, diag + f2 * wz * wz, tz,
                )
                for kk, v in enumerate(cols):
                    rr = jnp.full((_L,), kk // 4, jnp.int32)
                    cc = jnp.full((_L,), kk % 4, jnp.int32)
                    plsc.store_scatter(out_v, [orow, rr, cc], v)
        pltpu.sync_copy(out_v, out_hbm.at[pl.ds(base, bpw)])

    return k(indices, table)


def kernel(indices, pose_adjustment):
    b = indices.shape[0]
    table8 = jnp.pad(pose_adjustment, ((0, 0), (0, 2)))
    return _gather_expmap(indices, table8, b // _NW)


# speedup vs baseline: 1.8677x; 1.8677x over previous
import functools

import jax
import jax.numpy as jnp
from jax import lax
from jax.experimental import pallas as pl
from jax.experimental.pallas import tpu as pltpu
from jax.experimental.pallas import tpu_sc as plsc

_NC = 2
_NS = 16
_NW = _NC * _NS
_L = 16
_RPT = 64

_F1 = (1.0, -1.0 / 6, 1.0 / 120, -1.0 / 5040, 1.0 / 362880, -1.0 / 39916800)
_F2 = (0.5, -1.0 / 24, 1.0 / 720, -1.0 / 40320, 1.0 / 3628800,
       -1.0 / 479001600)


def _poly(coeffs, x):
    acc = jnp.full((_L,), coeffs[-1], jnp.float32)
    for c in coeffs[-2::-1]:
        acc = acc * x + jnp.float32(c)
    return acc


def _iota():
    return jnp.arange(_L, dtype=jnp.int32)


@functools.partial(jax.jit, static_argnums=(2,))
def _gather_expmap(indices, table8, bpw):
    b = indices.shape[0]
    nbt = b // 128
    tr = bpw // _RPT
    cpw = bpw // 128
    umax = table8.shape[0] - 1
    mesh = plsc.VectorSubcoreMesh(
        core_axis_name="c", subcore_axis_name="s",
        num_cores=_NC, num_subcores=_NS)

    @functools.partial(
        pl.kernel,
        mesh=mesh,
        out_type=jax.ShapeDtypeStruct((3, nbt, 4, 128), jnp.float32),
        scratch_types=[
            pltpu.VMEM((bpw,), jnp.int32),
            pltpu.VMEM((tr, 2 * _RPT), jnp.int32),
            pltpu.VMEM((tr, 2 * _RPT, 8), jnp.float32),
            pltpu.VMEM((3, cpw, 4, 128), jnp.float32),
            pltpu.SemaphoreType.DMA,
        ],
        compiler_params=pltpu.CompilerParams(
            needs_layout_passes=False, use_tc_tiling_on_sc=False),
    )
    def k(idx_hbm, table_hbm, out_hbm, idx_v, units_v, rows_v, out_v, sem):
        wid = lax.axis_index("s") * _NC + lax.axis_index("c")
        pltpu.sync_copy(idx_hbm.at[pl.ds(wid * bpw, bpw)], idx_v)
        copies = []
        for j in range(tr):
            jidx = jnp.full((_L,), j, jnp.int32)
            for s in range(_RPT // _L):
                idx16 = idx_v[pl.ds(j * _RPT + s * _L, _L)]
                u0 = (idx16 * 6) >> 3
                u1 = jnp.minimum(u0 + 1, jnp.int32(umax))
                pos = _iota() * 2 + (s * 2 * _L)
                plsc.store_scatter(units_v, [jidx, pos], u0)
                plsc.store_scatter(units_v, [jidx, pos + 1], u1)
            copies.append(pltpu.async_copy(
                table_hbm.at[units_v.at[j]], rows_v.at[j], sem))
        for j in range(tr):
            copies[j].wait()
            jidx = jnp.full((_L,), j, jnp.int32)
            for s in range(_RPT // _L):
                idx16 = idx_v[pl.ds(j * _RPT + s * _L, _L)]
                wb = _iota() * 16 + ((idx16 * 6) & 7) + (s * 16 * _L)

                def col(kk):
                    w = wb + kk
                    return plsc.load_gather(
                        rows_v, [jidx, w >> 3, w & 7])

                tx, ty, tz = col(0), col(1), col(2)
                wx, wy, wz = col(3), col(4), col(5)
                nr = wx * wx + wy * wy + wz * wz
                x = jnp.maximum(nr, jnp.float32(1e-4))
                f1 = _poly(_F1, x)
                f2 = _poly(_F2, x)
                diag = 1.0 - f2 * nr
                cwx = f2 * wx
                cxy = cwx * wy
                cxz = cwx * wz
                cyz = f2 * wy * wz
                sx = f1 * wx
                sy = f1 * wy
                sz = f1 * wz
                cols = (
                    diag + cwx * wx, cxy - sz, cxz + sy, tx,
                    cxy + sz, diag + f2 * wy * wy, cyz - sx, ty,
                    cxz - sy, cyz + sx, diag + f2 * wz * wz, tz,
                )
                gbase = j * _RPT + s * _L
                jt, bl = gbase // 128, gbase % 128
                for kk, v in enumerate(cols):
                    out_v[kk // 4, jt, kk % 4, pl.ds(bl, _L)] = v
        for r in range(3):
            pltpu.sync_copy(out_v.at[r], out_hbm.at[r, pl.ds(wid * cpw, cpw)])

    return k(indices, table8)


def kernel(indices, pose_adjustment):
    b = indices.shape[0]
    v, d = pose_adjustment.shape
    table8 = pose_adjustment.reshape(v * d // 8, 8)
    out4 = _gather_expmap(indices, table8, b // _NW)
    return lax.reshape(out4, (b, 3, 4), dimensions=(1, 3, 0, 2))


# speedup vs baseline: 5.6393x; 3.0194x over previous
import functools

import jax
import jax.numpy as jnp
from jax import lax
from jax.experimental import pallas as pl
from jax.experimental.pallas import tpu as pltpu
from jax.experimental.pallas import tpu_sc as plsc

_NC = 2
_NS = 16
_NW = _NC * _NS
_L = 16
_CH = 128

_F1 = (1.0, -1.0 / 6, 1.0 / 120, -1.0 / 5040, 1.0 / 362880, -1.0 / 39916800)
_F2 = (0.5, -1.0 / 24, 1.0 / 720, -1.0 / 40320, 1.0 / 3628800,
       -1.0 / 479001600)


def _poly(coeffs, x):
    acc = jnp.full((_L,), coeffs[-1], jnp.float32)
    for c in coeffs[-2::-1]:
        acc = acc * x + jnp.float32(c)
    return acc


def _iota():
    return jnp.arange(_L, dtype=jnp.int32)


@functools.partial(jax.jit, static_argnums=(2, 3))
def _gather_expmap(indices, table8, bpw, vrows):
    b = indices.shape[0]
    nbt = b // 128
    cpw = bpw // _CH
    ustride = vrows // 8
    mesh = plsc.VectorSubcoreMesh(
        core_axis_name="c", subcore_axis_name="s",
        num_cores=_NC, num_subcores=_NS)

    @functools.partial(
        pl.kernel,
        mesh=mesh,
        out_type=jax.ShapeDtypeStruct((3, nbt, 4, 128), jnp.float32),
        scratch_types=[
            pltpu.VMEM((bpw,), jnp.int32),
            pltpu.VMEM((cpw * 6, _CH), jnp.int32),
            pltpu.VMEM((cpw * 6, _CH, 8), jnp.float32),
            pltpu.VMEM((3, cpw, 4, 128), jnp.float32),
            pltpu.SemaphoreType.DMA,
        ],
        compiler_params=pltpu.CompilerParams(
            needs_layout_passes=False, use_tc_tiling_on_sc=False),
    )
    def k(idx_hbm, table_hbm, out_hbm, idx_v, units_v, rows_v, out_v, sem):
        wid = lax.axis_index("s") * _NC + lax.axis_index("c")
        pltpu.sync_copy(idx_hbm.at[pl.ds(wid * bpw, bpw)], idx_v)
        copies = []
        for j in range(cpw):
            for s in range(_CH // _L):
                idx16 = idx_v[pl.ds(j * _CH + s * _L, _L)]
                u = idx16 >> 3
                pos = _iota() + (s * _L)
                for c in range(6):
                    plsc.store_scatter(
                        units_v, [jnp.full((_L,), j * 6 + c, jnp.int32), pos],
                        u + (c * ustride))
            for c in range(6):
                jj = j * 6 + c
                copies.append(pltpu.async_copy(
                    table_hbm.at[units_v.at[jj]], rows_v.at[jj], sem))
        for j in range(cpw):
            for c in range(6):
                copies[j * 6 + c].wait()
            for s in range(_CH // _L):
                idx16 = idx_v[pl.ds(j * _CH + s * _L, _L)]
                off = idx16 & 7
                row = _iota() + (s * _L)

                def col(kk):
                    return plsc.load_gather(
                        rows_v,
                        [jnp.full((_L,), j * 6 + kk, jnp.int32), row, off])

                tx, ty, tz = col(0), col(1), col(2)
                wx, wy, wz = col(3), col(4), col(5)
                nr = wx * wx + wy * wy + wz * wz
                x = jnp.maximum(nr, jnp.float32(1e-4))
                f1 = _poly(_F1, x)
                f2 = _poly(_F2, x)
                diag = 1.0 - f2 * nr
                cwx = f2 * wx
                cxy = cwx * wy
                cxz = cwx * wz
                cyz = f2 * wy * wz
                sx = f1 * wx
                sy = f1 * wy
                sz = f1 * wz
                cols = (
                    diag + cwx * wx, cxy - sz, cxz + sy, tx,
                    cxy + sz, diag + f2 * wy * wy, cyz - sx, ty,
                    cxz - sy, cyz + sx, diag + f2 * wz * wz, tz,
                )
                for kk, v in enumerate(cols):
                    out_v[kk // 4, j, kk % 4, pl.ds(s * _L, _L)] = v
        for r in range(3):
            pltpu.sync_copy(out_v.at[r], out_hbm.at[r, pl.ds(wid * cpw, cpw)])

    return k(indices, table8)


def kernel(indices, pose_adjustment):
    b = indices.shape[0]
    v, d = pose_adjustment.shape
    table8 = pose_adjustment.T.reshape(v * d // 8, 8)
    out4 = _gather_expmap(indices, table8, b // _NW, v)
    return lax.reshape(out4, (b, 3, 4), dimensions=(1, 3, 0, 2))


# speedup vs baseline: 5.8969x; 1.0457x over previous
import functools

import jax
import jax.numpy as jnp
from jax import lax
from jax.experimental import pallas as pl
from jax.experimental.pallas import tpu as pltpu
from jax.experimental.pallas import tpu_sc as plsc

_NC = 2
_NS = 16
_NW = _NC * _NS
_L = 16
_CH = 128

_F1 = (1.0, -1.0 / 6, 1.0 / 120, -1.0 / 5040, 1.0 / 362880, -1.0 / 39916800)
_F2 = (0.5, -1.0 / 24, 1.0 / 720, -1.0 / 40320, 1.0 / 3628800,
       -1.0 / 479001600)


def _poly(coeffs, x):
    acc = jnp.full((_L,), coeffs[-1], jnp.float32)
    for c in coeffs[-2::-1]:
        acc = acc * x + jnp.float32(c)
    return acc


def _iota():
    return jnp.arange(_L, dtype=jnp.int32)


@functools.partial(jax.jit, static_argnums=(2,))
def _gather_expmap(indices, table3, bpw):
    b = indices.shape[0]
    nbt = b // 128
    cpw = bpw // _CH
    mesh = plsc.VectorSubcoreMesh(
        core_axis_name="c", subcore_axis_name="s",
        num_cores=_NC, num_subcores=_NS)

    @functools.partial(
        pl.kernel,
        mesh=mesh,
        out_type=jax.ShapeDtypeStruct((3, nbt, 4, 128), jnp.float32),
        scratch_types=[
            pltpu.VMEM((bpw,), jnp.int32),
            pltpu.VMEM((cpw, _CH), jnp.int32),
            pltpu.VMEM((cpw * 6, _CH, 8), jnp.float32),
            pltpu.VMEM((3, cpw, 4, 128), jnp.float32),
            pltpu.SemaphoreType.DMA,
        ],
        compiler_params=pltpu.CompilerParams(
            needs_layout_passes=False, use_tc_tiling_on_sc=False),
    )
    def k(idx_hbm, table_hbm, out_hbm, idx_v, units_v, rows_v, out_v, sem):
        wid = lax.axis_index("s") * _NC + lax.axis_index("c")
        pltpu.sync_copy(idx_hbm.at[pl.ds(wid * bpw, bpw)], idx_v)
        copies = []
        for j in range(cpw):
            for s in range(_CH // _L):
                idx16 = idx_v[pl.ds(j * _CH + s * _L, _L)]
                units_v[j, pl.ds(s * _L, _L)] = idx16 >> 3
            for c in range(6):
                copies.append(pltpu.async_copy(
                    table_hbm.at[c].at[units_v.at[j]],
                    rows_v.at[j * 6 + c], sem))
        for j in range(cpw):
            for c in range(6):
                copies[j * 6 + c].wait()
            for s in range(_CH // _L):
                idx16 = idx_v[pl.ds(j * _CH + s * _L, _L)]
                off = idx16 & 7
                row = _iota() + (s * _L)

                def col(kk):
                    return plsc.load_gather(
                        rows_v,
                        [jnp.full((_L,), j * 6 + kk, jnp.int32), row, off])

                tx, ty, tz = col(0), col(1), col(2)
                wx, wy, wz = col(3), col(4), col(5)
                nr = wx * wx + wy * wy + wz * wz
                x = jnp.maximum(nr, jnp.float32(1e-4))
                f1 = _poly(_F1, x)
                f2 = _poly(_F2, x)
                diag = 1.0 - f2 * nr
                cwx = f2 * wx
                cxy = cwx * wy
                cxz = cwx * wz
                cyz = f2 * wy * wz
                sx = f1 * wx
                sy = f1 * wy
                sz = f1 * wz
                cols = (
                    diag + cwx * wx, cxy - sz, cxz + sy, tx,
                    cxy + sz, diag + f2 * wy * wy, cyz - sx, ty,
                    cxz - sy, cyz + sx, diag + f2 * wz * wz, tz,
                )
                for kk, v in enumerate(cols):
                    out_v[kk // 4, j, kk % 4, pl.ds(s * _L, _L)] = v
        for r in range(3):
            pltpu.sync_copy(out_v.at[r], out_hbm.at[r, pl.ds(wid * cpw, cpw)])

    return k(indices, table3)


def kernel(indices, pose_adjustment):
    b = indices.shape[0]
    v, d = pose_adjustment.shape
    table3 = pose_adjustment.T.reshape(d, v // 8, 8)
    out4 = _gather_expmap(indices, table3, b // _NW)
    return lax.reshape(out4, (b, 3, 4), dimensions=(1, 3, 0, 2))


# speedup vs baseline: 5.9092x; 1.0021x over previous
import functools

import jax
import jax.numpy as jnp
from jax import lax
from jax.experimental import pallas as pl
from jax.experimental.pallas import tpu as pltpu
from jax.experimental.pallas import tpu_sc as plsc

_NC = 2
_NS = 16
_NW = _NC * _NS
_L = 16
_CH = 128

_F1 = (1.0, -1.0 / 6, 1.0 / 120, -1.0 / 5040, 1.0 / 362880, -1.0 / 39916800)
_F2 = (0.5, -1.0 / 24, 1.0 / 720, -1.0 / 40320, 1.0 / 3628800,
       -1.0 / 479001600)


def _poly(coeffs, x):
    acc = jnp.full((_L,), coeffs[-1], jnp.float32)
    for c in coeffs[-2::-1]:
        acc = acc * x + jnp.float32(c)
    return acc


def _iota():
    return jnp.arange(_L, dtype=jnp.int32)


@functools.partial(jax.jit, static_argnums=(2,))
def _gather_expmap(indices, table3, bpw):
    b = indices.shape[0]
    nbt = b // 128
    cpw = bpw // _CH
    mesh = plsc.VectorSubcoreMesh(
        core_axis_name="c", subcore_axis_name="s",
        num_cores=_NC, num_subcores=_NS)

    @functools.partial(
        pl.kernel,
        mesh=mesh,
        out_type=jax.ShapeDtypeStruct((3, nbt, 4, 128), jnp.float32),
        scratch_types=[
            pltpu.VMEM((bpw,), jnp.int32),
            pltpu.VMEM((cpw, _CH), jnp.int32),
            pltpu.VMEM((cpw * 6, _CH, 8), jnp.float32),
            pltpu.VMEM((3, cpw, 4, 128), jnp.float32),
            pltpu.SemaphoreType.DMA,
        ],
        compiler_params=pltpu.CompilerParams(
            needs_layout_passes=False, use_tc_tiling_on_sc=False),
    )
    def k(idx_hbm, table_hbm, out_hbm, idx_v, units_v, rows_v, out_v, sem):
        wid = lax.axis_index("s") * _NC + lax.axis_index("c")
        pltpu.sync_copy(idx_hbm.at[pl.ds(wid * bpw, bpw)], idx_v)
        copies = []
        for j in range(cpw):
            for s in range(_CH // _L):
                idx16 = idx_v[pl.ds(j * _CH + s * _L, _L)]
                units_v[j, pl.ds(s * _L, _L)] = idx16 >> 3
            for c in range(6):
                copies.append(pltpu.async_copy(
                    table_hbm.at[c].at[units_v.at[j]],
                    rows_v.at[j * 6 + c], sem))
        for j in range(cpw):
            for c in range(6):
                copies[j * 6 + c].wait()
            for s in range(_CH // _L):
                idx16 = idx_v[pl.ds(j * _CH + s * _L, _L)]
                off = idx16 & 7
                row = _iota() + (s * _L)

                def col(kk):
                    return plsc.load_gather(
                        rows_v,
                        [jnp.full((_L,), j * 6 + kk, jnp.int32), row, off])

                tx, ty, tz = col(0), col(1), col(2)
                wx, wy, wz = col(3), col(4), col(5)
                nr = wx * wx + wy * wy + wz * wz
                x = jnp.maximum(nr, jnp.float32(1e-4))
                f1 = _poly(_F1, x)
                f2 = _poly(_F2, x)
                diag = 1.0 - f2 * nr
                cwx = f2 * wx
                cxy = cwx * wy
                cxz = cwx * wz
                cyz = f2 * wy * wz
                sx = f1 * wx
                sy = f1 * wy
                sz = f1 * wz
                cols = (
                    diag + cwx * wx, cxy - sz, cxz + sy, tx,
                    cxy + sz, diag + f2 * wy * wy, cyz - sx, ty,
                    cxz - sy, cyz + sx, diag + f2 * wz * wz, tz,
                )
                for kk, v in enumerate(cols):
                    out_v[kk // 4, j, kk % 4, pl.ds(s * _L, _L)] = v
        outs = [
            pltpu.async_copy(
                out_v.at[r], out_hbm.at[r, pl.ds(wid * cpw, cpw)], sem)
            for r in range(3)
        ]
        for o in outs:
            o.wait()

    return k(indices, table3)


def kernel(indices, pose_adjustment):
    b = indices.shape[0]
    v, d = pose_adjustment.shape
    table3 = pose_adjustment.T.reshape(d, v // 8, 8)
    out4 = _gather_expmap(indices, table3, b // _NW)
    return lax.reshape(out4, (b, 3, 4), dimensions=(1, 3, 0, 2))
